# Initial kernel scaffold; baseline (speedup 1.0000x reference)
import jax
import jax.numpy as jnp
from jax.experimental import pallas as pl


def kernel(x_operator, edge_index_calledby, batch_operator, W_in, b_in, Wk, bk, Wq, bq, Wv, bv, a_rel, m_rel, p_rel, Wa, ba, skip, ln_g, ln_b, Wm, bm, Wt, bt):
    raise NotImplementedError("write your pallas kernel here")


# speedup vs baseline: 18.8543x; 18.8543x over previous
import functools

import jax
import jax.numpy as jnp
from jax import lax
from jax.experimental import pallas as pl
from jax.experimental.pallas import tpu as pltpu
from jax.experimental.pallas import tpu_sc as plsc

N = 50000
E = 800000
F_IN = 128
C = 64
H = 4
B = 64
D = C // H

R = 2000
NBLK = N // R
RE = 2000
EBLK = E // RE
NTILE = 16
CH = 128
NCH = E // CH
NPA = 12544
AROWS_PT = NPA // NTILE
NPD = 1664
DROWS_PT = NPD // NTILE


def _pre1_body(x_ref, Win_ref, bin_ref, Wq_ref, bq_ref, Wk_ref, bk_ref,
               Wv_ref, bv_ref, h_ref, qp_ref, kv_ref):
  h = jnp.dot(x_ref[...], Win_ref[...], preferred_element_type=jnp.float32)
  h = h + bin_ref[...]
  h_ref[...] = h
  q = jnp.dot(h, Wq_ref[...], preferred_element_type=jnp.float32) + bq_ref[...]
  k = jnp.dot(h, Wk_ref[...], preferred_element_type=jnp.float32) + bk_ref[...]
  v = jnp.dot(h, Wv_ref[...], preferred_element_type=jnp.float32) + bv_ref[...]
  qp_ref[...] = jnp.concatenate([q, q], axis=1)
  kv_ref[...] = jnp.concatenate([k, v], axis=1)


def _w_spec(shape):
  return pl.BlockSpec(shape, lambda i: (0,) * len(shape))


_row_spec64 = pl.BlockSpec((R, C), lambda i: (i, 0))
_row_spec128 = pl.BlockSpec((R, 128), lambda i: (i, 0))


def _tc_pre1(x, Win, bin_, Wq, bq, Wk, bk, Wv, bv):
  out = jax.ShapeDtypeStruct((N, 128), jnp.float32)
  return pl.pallas_call(
      _pre1_body,
      grid=(NBLK,),
      in_specs=[
          pl.BlockSpec((R, F_IN), lambda i: (i, 0)),
          _w_spec((F_IN, C)), _w_spec((1, C)),
          _w_spec((C, C)), _w_spec((1, C)),
          _w_spec((C, C)), _w_spec((1, C)),
          _w_spec((C, C)), _w_spec((1, C)),
      ],
      out_specs=[_row_spec64, _row_spec128, _row_spec128],
      out_shape=[jax.ShapeDtypeStruct((N, C), jnp.float32), out, out],
  )(x, Win, bin_, Wq, bq, Wk, bk, Wv, bv)


def _qkv_body(h_ref, Wq_ref, bq_ref, Wk_ref, bk_ref, Wv_ref, bv_ref,
              qp_ref, kv_ref):
  h = h_ref[...]
  q = jnp.dot(h, Wq_ref[...], preferred_element_type=jnp.float32) + bq_ref[...]
  k = jnp.dot(h, Wk_ref[...], preferred_element_type=jnp.float32) + bk_ref[...]
  v = jnp.dot(h, Wv_ref[...], preferred_element_type=jnp.float32) + bv_ref[...]
  qp_ref[...] = jnp.concatenate([q, q], axis=1)
  kv_ref[...] = jnp.concatenate([k, v], axis=1)


def _tc_qkv(h, Wq, bq, Wk, bk, Wv, bv):
  out = jax.ShapeDtypeStruct((N, 128), jnp.float32)
  return pl.pallas_call(
      _qkv_body,
      grid=(NBLK,),
      in_specs=[
          _row_spec64,
          _w_spec((C, C)), _w_spec((1, C)),
          _w_spec((C, C)), _w_spec((1, C)),
          _w_spec((C, C)), _w_spec((1, C)),
      ],
      out_specs=[_row_spec128, _row_spec128],
      out_shape=[out, out],
  )(h, Wq, bq, Wk, bk, Wv, bv)


def _make_sc_gather():
  mesh = plsc.VectorSubcoreMesh(core_axis_name="c", subcore_axis_name="s")
  eout = jax.ShapeDtypeStruct((E, 128), jnp.float32)

  @functools.partial(
      pl.kernel,
      mesh=mesh,
      out_type=[eout, eout],
      scratch_types=[
          pltpu.VMEM((CH,), jnp.int32),
          pltpu.VMEM((CH,), jnp.int32),
          pltpu.VMEM((CH, 128), jnp.float32),
          pltpu.VMEM((CH, 128), jnp.float32),
          pltpu.SemaphoreType.DMA,
      ],
  )
  def sc_gather(qp_hbm, kv_hbm, src_hbm, dst_hbm,
                qd_out, kvs_out,
                src_v, dst_v, qb, kvb, sem):
    c = lax.axis_index("c")
    s = lax.axis_index("s")
    wid = s * 2 + c
    nw = (NCH - wid + 31) // 32

    def chunk(i, _):
      e0 = (wid + 32 * i) * CH
      pltpu.sync_copy(src_hbm.at[pl.ds(e0, CH)], src_v)
      pltpu.sync_copy(dst_hbm.at[pl.ds(e0, CH)], dst_v)
      cp_q = pltpu.async_copy(qp_hbm.at[dst_v], qb, sem)
      cp_kv = pltpu.async_copy(kv_hbm.at[src_v], kvb, sem)
      cp_q.wait()
      cp_kv.wait()
      pltpu.sync_copy(qb, qd_out.at[pl.ds(e0, CH)])
      pltpu.sync_copy(kvb, kvs_out.at[pl.ds(e0, CH)])
      return ()

    lax.fori_loop(0, nw, chunk, ())

  return sc_gather


_sc_gather = _make_sc_gather()


def _mid_body(qd_ref, kvs_ref, p32_ref, p16_ref, wv0_ref, wv1_ref, exd_ref):
  qd = qd_ref[...]
  kvs = kvs_ref[...]
  prod = qd[:, 0:C] * kvs[:, 0:C]
  exs = []
  vws = []
  for h in range(H):
    alpha = jnp.sum(prod[:, h * D:(h + 1) * D], axis=1, keepdims=True)
    ex = jnp.exp(alpha)
    exs.append(ex)
    vws.append(kvs[:, C + h * D:C + (h + 1) * D] * ex)

  p32 = p32_ref[...]
  zero32 = jnp.zeros((RE, 32), jnp.float32)
  for cc, ref in ((0, wv0_ref), (1, wv1_ref)):
    base = jnp.concatenate(vws[2 * cc:2 * cc + 2], axis=1)
    out = jnp.zeros((RE, 128), jnp.float32)
    for m in range(4):
      variant = jnp.concatenate(
          [zero32] * m + [base] + [zero32] * (3 - m), axis=1)
      out = jnp.where(p32 == 32 * m, variant, out)
    ref[...] = out

  p16 = p16_ref[...]
  colidx = lax.broadcasted_iota(jnp.int32, (RE, 128), 1)
  exd = jnp.zeros((RE, 128), jnp.float32)
  for h in range(H):
    exd = jnp.where(colidx == p16 + h,
                    jnp.broadcast_to(exs[h], (RE, 128)), exd)
  exd_ref[...] = exd


def _tc_mid(qd, kvs, p32, p16):
  espec = pl.BlockSpec((RE, 128), lambda i: (i, 0))
  iospec = pl.BlockSpec((RE, 1), lambda i: (i, 0))
  eout = jax.ShapeDtypeStruct((E, 128), jnp.float32)
  return pl.pallas_call(
      _mid_body,
      grid=(EBLK,),
      in_specs=[espec, espec, iospec, iospec],
      out_specs=[espec, espec, espec],
      out_shape=[eout, eout, eout],
  )(qd, kvs, p32, p16)


def _make_sc_scatter_v():
  mesh = plsc.VectorSubcoreMesh(core_axis_name="c", subcore_axis_name="s")

  @functools.partial(
      pl.kernel,
      mesh=mesh,
      out_type=jax.ShapeDtypeStruct((2 * NPA, 128), jnp.float32),
      scratch_types=[
          pltpu.VMEM((CH,), jnp.int32),
          pltpu.VMEM((CH,), jnp.int32),
          pltpu.VMEM((CH, 128), jnp.float32),
          pltpu.VMEM_SHARED((NPA, 128), jnp.float32),
      ],
  )
  def sc_scatter_v(wv0_hbm, wv1_hbm, dst_hbm, za_hbm,
                   exv_out, dst_v, dsh_v, wvb, acc):
    c = lax.axis_index("c")
    s = lax.axis_index("s")
    r0 = s * AROWS_PT

    pltpu.sync_copy(za_hbm, acc.at[pl.ds(r0, AROWS_PT)])
    plsc.subcore_barrier()

    ns = (NCH - s + 15) // 16

    def chunk(i, _):
      e0 = (s + 16 * i) * CH
      pltpu.sync_copy(dst_hbm.at[pl.ds(e0, CH)], dst_v)
      for g in range(CH // 16):
        dvg = dst_v[pl.ds(g * 16, 16)]
        dsh_v[pl.ds(g * 16, 16)] = lax.shift_right_logical(dvg, 2)

      @pl.when(c == 0)
      def _():
        pltpu.sync_copy(wv0_hbm.at[pl.ds(e0, CH)], wvb)

      @pl.when(c == 1)
      def _():
        pltpu.sync_copy(wv1_hbm.at[pl.ds(e0, CH)], wvb)

      pltpu.sync_copy(wvb, acc.at[dsh_v], add=True)
      return ()

    lax.fori_loop(0, ns, chunk, ())
    plsc.subcore_barrier()

    o0 = c * NPA + r0
    pltpu.sync_copy(acc.at[pl.ds(r0, AROWS_PT)], exv_out.at[pl.ds(o0, AROWS_PT)])

  return sc_scatter_v


def _make_sc_scatter_d():
  mesh = plsc.VectorSubcoreMesh(core_axis_name="c", subcore_axis_name="s")

  @functools.partial(
      pl.kernel,
      mesh=mesh,
      out_type=jax.ShapeDtypeStruct((2 * NPD, 128), jnp.float32),
      scratch_types=[
          pltpu.VMEM((CH,), jnp.int32),
          pltpu.VMEM((CH,), jnp.int32),
          pltpu.VMEM((CH, 128), jnp.float32),
          pltpu.VMEM_SHARED((NPD, 128), jnp.float32),
      ],
  )
  def sc_scatter_d(exd_hbm, dst_hbm, zd_hbm,
                   exd_out, dst_v, dsh_v, exdb, accD):
    c = lax.axis_index("c")
    s = lax.axis_index("s")
    rD0 = s * DROWS_PT

    pltpu.sync_copy(zd_hbm, accD.at[pl.ds(rD0, DROWS_PT)])
    plsc.subcore_barrier()

    nd = (NCH // 2 - s + 15) // 16

    def chunkd(i, _):
      e0 = (c * (NCH // 2) + s + 16 * i) * CH
      pltpu.sync_copy(dst_hbm.at[pl.ds(e0, CH)], dst_v)
      for g in range(CH // 16):
        dvg = dst_v[pl.ds(g * 16, 16)]
        dsh_v[pl.ds(g * 16, 16)] = lax.shift_right_logical(dvg, 5)
      pltpu.sync_copy(exd_hbm.at[pl.ds(e0, CH)], exdb)
      pltpu.sync_copy(exdb, accD.at[dsh_v], add=True)
      return ()

    lax.fori_loop(0, nd, chunkd, ())
    plsc.subcore_barrier()

    oD0 = c * NPD + rD0
    pltpu.sync_copy(accD.at[pl.ds(rD0, DROWS_PT)],
                    exd_out.at[pl.ds(oD0, DROWS_PT)])

  return sc_scatter_d


_sc_scatter_v = _make_sc_scatter_v()
_sc_scatter_d = _make_sc_scatter_d()


def _post_body(exv_ref, exd_ref, h_ref, Wa_ref, ba_ref, skip_ref,
               lng_ref, lnb_ref, ho_ref):
  out64 = jnp.concatenate([exv_ref[0], exv_ref[1]], axis=-1)
  exd = exd_ref[0] + exd_ref[1]
  dens = []
  for m in range(H):
    dens.append(jnp.broadcast_to(exd[:, m:m + 1], (R, D)))
  den64 = jnp.concatenate(dens, axis=-1)
  out = out64 / (den64 + 1e-16)
  out = 0.5 * out * (1.0 + lax.erf(out * 0.7071067811865476))
  out = jnp.dot(out, Wa_ref[...], preferred_element_type=jnp.float32) + ba_ref[...]
  beta = 1.0 / (1.0 + jnp.exp(-skip_ref[0, 0]))
  h = beta * out + (1.0 - beta) * h_ref[...]
  e = jnp.where(h > 0, h, jnp.exp(jnp.minimum(h, 0.0)) - 1.0)
  mu = jnp.mean(e, axis=-1, keepdims=True)
  var = jnp.mean((e - mu) ** 2, axis=-1, keepdims=True)
  ho_ref[...] = (e - mu) * lax.rsqrt(var + 1e-5) * lng_ref[...] + lnb_ref[...]


def _tc_post(exv, exd, h, Wa, ba, skip, lng, lnb):
  return pl.pallas_call(
      _post_body,
      grid=(NBLK,),
      in_specs=[
          pl.BlockSpec((2, R, 32), lambda i: (0, i, 0)),
          pl.BlockSpec((2, R, 4), lambda i: (0, i, 0)),
          _row_spec64,
          _w_spec((C, C)), _w_spec((1, C)),
          _w_spec((1, 1)),
          _w_spec((1, C)), _w_spec((1, C)),
      ],
      out_specs=_row_spec64,
      out_shape=jax.ShapeDtypeStruct((N, C), jnp.float32),
  )(exv, exd, h, Wa, ba, skip, lng, lnb)


def _pool_body(h_ref, bo_ref, Wmt_ref, bmt_ref, o_ref, acc):
  i = pl.program_id(0)

  @pl.when(i == 0)
  def _():
    acc[...] = jnp.zeros_like(acc)

  ids = bo_ref[0]
  rows = lax.broadcasted_iota(jnp.int32, (B, R), 0)
  oh = (rows == ids).astype(jnp.float32)
  h_aug = jnp.concatenate(
      [h_ref[...], jnp.ones((R, 1), jnp.float32), jnp.zeros((R, 63), jnp.float32)],
      axis=-1)
  acc[...] += jnp.dot(oh, h_aug, preferred_element_type=jnp.float32)

  @pl.when(i == NBLK - 1)
  def _():
    sums = acc[:, :C]
    cnt = acc[:, C:C + 1]
    emb = sums / jnp.maximum(cnt, 1.0)
    o_ref[...] = jnp.dot(emb, Wmt_ref[...], preferred_element_type=jnp.float32) + bmt_ref[...]


def _tc_pool(h, bo3, Wmt, bmt):
  return pl.pallas_call(
      _pool_body,
      grid=(NBLK,),
      in_specs=[
          _row_spec64,
          pl.BlockSpec((1, 1, R), lambda i: (i, 0, 0)),
          _w_spec((C, 128)), _w_spec((1, 128)),
      ],
      out_specs=_w_spec((B, 128)),
      out_shape=jax.ShapeDtypeStruct((B, 128), jnp.float32),
      scratch_shapes=[pltpu.VMEM((B, 128), jnp.float32)],
  )(h, bo3, Wmt, bmt)


def kernel(x_operator, edge_index_calledby, batch_operator, W_in, b_in,
           Wk, bk, Wq, bq, Wv, bv, a_rel, m_rel, p_rel, Wa, ba, skip,
           ln_g, ln_b, Wm, bm, Wt, bt):
  f32 = jnp.float32
  scale = (p_rel / jnp.sqrt(jnp.float32(D))).astype(f32)
  Wq2 = (Wq.reshape(C, H, D) * scale[None, :, None]).reshape(C, C)
  bq2 = (bq.reshape(H, D) * scale[:, None]).reshape(C)
  Wk2 = jnp.einsum('nhd,hde->nhe', Wk.reshape(C, H, D), a_rel).reshape(C, C)
  bk2 = jnp.einsum('hd,hde->he', bk.reshape(H, D), a_rel).reshape(C)
  Wv2 = jnp.einsum('nhd,hde->nhe', Wv.reshape(C, H, D), m_rel).reshape(C, C)
  bv2 = jnp.einsum('hd,hde->he', bv.reshape(H, D), m_rel).reshape(C)

  r1 = lambda a: a.reshape(1, -1)
  src = edge_index_calledby[0]
  dst = edge_index_calledby[1]
  p32 = ((dst & 3) * 32).astype(jnp.int32).reshape(E, 1)
  p16 = ((dst & 31) * 4).astype(jnp.int32).reshape(E, 1)
  za = jnp.zeros((AROWS_PT, 128), f32)
  zd = jnp.zeros((DROWS_PT, 128), f32)
  skip2 = skip.reshape(1, 1)

  h, qp, kv = _tc_pre1(x_operator, W_in, r1(b_in), Wq2, r1(bq2),
                       Wk2, r1(bk2), Wv2, r1(bv2))
  for layer in range(2):
    qd, kvs = _sc_gather(qp, kv, src, dst)
    wv0, wv1, exd = _tc_mid(qd, kvs, p32, p16)
    exv = _sc_scatter_v(wv0, wv1, dst, za)
    exdacc = _sc_scatter_d(exd, dst, zd)
    h = _tc_post(exv.reshape(2, 4 * NPA, 32)[:, :N],
                 exdacc.reshape(2, 32 * NPD, 4)[:, :N],
                 h, Wa, r1(ba), skip2, r1(ln_g), r1(ln_b))
    if layer == 0:
      qp, kv = _tc_qkv(h, Wq2, r1(bq2), Wk2, r1(bk2), Wv2, r1(bv2))

  Wmt = jnp.zeros((C, 128), f32).at[:, 0].set(Wm[:, 0]).at[:, 1].set(Wt[:, 0])
  bmt = jnp.zeros((1, 128), f32).at[0, 0].set(bm[0]).at[0, 1].set(bt[0])
  bo3 = batch_operator.reshape(NBLK, 1, R)
  res = _tc_pool(h, bo3, Wmt, bmt)
  return res[:, 0], res[:, 1]
